# Initial kernel scaffold; baseline (speedup 1.0000x reference)
import jax
import jax.numpy as jnp
from jax.experimental import pallas as pl


def kernel(x, edge_index, edge_attr, params):
    raise NotImplementedError("write your pallas kernel here")


# speedup vs baseline: 3.0105x; 3.0105x over previous
import functools

import jax
import jax.numpy as jnp
from jax import lax
from jax.experimental import pallas as pl
from jax.experimental.pallas import tpu as pltpu
from jax.experimental.pallas import tpu_sc as plsc

_f32 = jnp.float32
_CH = 80


def _sc_mesh_info():
    info = plsc.get_sparse_core_info()
    return info.num_cores, info.num_subcores


def _gather_pair(x, src, dst):
    n, d = x.shape
    e = src.shape[0]
    nc, ns = _sc_mesh_info()
    nw = nc * ns
    ew = e // nw
    nch = ew // _CH
    mesh = plsc.VectorSubcoreMesh(core_axis_name="c", subcore_axis_name="s")

    def body(x_hbm, src_hbm, dst_hbm, xs_hbm, xd_hbm, idx_v, rows_v, sem):
        c = lax.axis_index("c")
        s = lax.axis_index("s")
        base = (s * nc + c) * ew

        def step(i, carry):
            off = pl.multiple_of(base + i * _CH, 8)
            pltpu.sync_copy(src_hbm.at[pl.ds(off, _CH)], idx_v)
            pltpu.async_copy(x_hbm.at[idx_v], rows_v, sem).wait()
            pltpu.sync_copy(rows_v, xs_hbm.at[pl.ds(off, _CH)])
            pltpu.sync_copy(dst_hbm.at[pl.ds(off, _CH)], idx_v)
            pltpu.async_copy(x_hbm.at[idx_v], rows_v, sem).wait()
            pltpu.sync_copy(rows_v, xd_hbm.at[pl.ds(off, _CH)])
            return carry

        lax.fori_loop(0, nch, step, 0)

    f = pl.kernel(
        body,
        mesh=mesh,
        out_type=[jax.ShapeDtypeStruct((e, d), _f32),
                  jax.ShapeDtypeStruct((e, d), _f32)],
        scratch_types=[pltpu.VMEM((_CH,), jnp.int32),
                       pltpu.VMEM((_CH, d), _f32),
                       pltpu.SemaphoreType.DMA],
        compiler_params=pltpu.CompilerParams(use_tc_tiling_on_sc=False),
    )
    return f(x, src, dst)


def _segment_parts(m, dst, n, with_cnt):
    e, z = m.shape
    nc, ns = _sc_mesh_info()
    nw = nc * ns
    ew = e // nw
    nch = ew // _CH
    rpt = -(-n // (ns * 8)) * 8
    npad = rpt * ns
    mesh = plsc.VectorSubcoreMesh(core_axis_name="c", subcore_axis_name="s")

    out_type = [jax.ShapeDtypeStruct((nc * npad, z), _f32)]
    scratch = [pltpu.VMEM((_CH,), jnp.int32),
               pltpu.VMEM((_CH, z), _f32),
               pltpu.VMEM_SHARED((npad, z), _f32)]
    if with_cnt:
        out_type.append(jax.ShapeDtypeStruct((nc * npad, 16), _f32))
        scratch += [pltpu.VMEM((_CH, 16), _f32),
                    pltpu.VMEM_SHARED((npad, 16), _f32)]

    def body(m_hbm, dst_hbm, zer_hbm, *rest):
        if with_cnt:
            (z16_hbm, one_hbm, agg_hbm, cnt_hbm,
             idx_v, m_v, acc_sh, one_v, cnt_sh) = rest
        else:
            agg_hbm, idx_v, m_v, acc_sh = rest
        c = lax.axis_index("c")
        s = lax.axis_index("s")
        base = (s * nc + c) * ew
        r0 = s * rpt
        pltpu.sync_copy(zer_hbm.at[pl.ds(r0, rpt)], acc_sh.at[pl.ds(r0, rpt)])
        if with_cnt:
            pltpu.sync_copy(z16_hbm.at[pl.ds(r0, rpt)], cnt_sh.at[pl.ds(r0, rpt)])
            pltpu.sync_copy(one_hbm, one_v)
        plsc.subcore_barrier()

        def step(i, carry):
            off = pl.multiple_of(base + i * _CH, 8)
            pltpu.sync_copy(dst_hbm.at[pl.ds(off, _CH)], idx_v)
            pltpu.sync_copy(m_hbm.at[pl.ds(off, _CH)], m_v)
            pltpu.sync_copy(m_v, acc_sh.at[idx_v], add=True)
            if with_cnt:
                pltpu.sync_copy(one_v, cnt_sh.at[idx_v], add=True)
            return carry

        lax.fori_loop(0, nch, step, 0)
        plsc.subcore_barrier()
        o0 = c * npad + r0
        pltpu.sync_copy(acc_sh.at[pl.ds(r0, rpt)], agg_hbm.at[pl.ds(o0, rpt)])
        if with_cnt:
            pltpu.sync_copy(cnt_sh.at[pl.ds(r0, rpt)], cnt_hbm.at[pl.ds(o0, rpt)])

    f = pl.kernel(body, mesh=mesh, out_type=out_type, scratch_types=scratch,
                  compiler_params=pltpu.CompilerParams(
                      use_tc_tiling_on_sc=False))
    zeros = jnp.zeros((npad, z), _f32)
    if with_cnt:
        agg, cnt = f(m, dst, zeros, jnp.zeros((npad, 16), _f32),
                     jnp.ones((_CH, 16), _f32))
        return (agg.reshape(nc, npad, z)[:, :n],
                cnt.reshape(nc, npad, 16)[:, :n])
    agg, = f(m, dst, zeros)
    return agg.reshape(nc, npad, z)[:, :n]


def _edge_mlp(xs, xd, ea, p, d):
    e = xs.shape[0]
    be = 3200
    na = ea.shape[1]
    W1 = p['We1']
    z = W1.shape[1]
    wi, wdf, wp = W1[0:d], W1[d:2 * d], W1[2 * d:3 * d]
    we = W1[3 * d:3 * d + 2]
    wea = W1[3 * d + 2:]
    b1 = p['be1'].reshape(1, z)
    w2 = p['We2']
    b2 = p['be2'].reshape(1, z)

    def body(xs_r, xd_r, ea_r, wi_r, wdf_r, wp_r, we_r, wea_r, b1_r, w2_r,
             b2_r, o_r):
        xj = xs_r[...]
        xi = xd_r[...]
        diff = xj - xi
        prod = xj * xi
        e1 = jnp.sqrt(jnp.sum(diff * diff, axis=1, keepdims=True) + 1e-12)
        e2 = jnp.sum(prod, axis=1, keepdims=True)
        acc = jnp.dot(xi, wi_r[...], preferred_element_type=_f32)
        acc = acc + jnp.dot(diff, wdf_r[...], preferred_element_type=_f32)
        acc = acc + jnp.dot(prod, wp_r[...], preferred_element_type=_f32)
        w = we_r[...]
        acc = acc + e1 * w[0:1, :] + e2 * w[1:2, :]
        acc = acc + jnp.dot(ea_r[...], wea_r[...], preferred_element_type=_f32)
        h = jnp.maximum(acc + b1_r[...], 0.0)
        o = jnp.dot(h, w2_r[...], preferred_element_type=_f32) + b2_r[...]
        o_r[...] = jnp.maximum(o, 0.0)

    full = lambda a: pl.BlockSpec(a.shape, lambda i: (0, 0))
    return pl.pallas_call(
        body,
        grid=(e // be,),
        in_specs=[pl.BlockSpec((be, d), lambda i: (i, 0)),
                  pl.BlockSpec((be, d), lambda i: (i, 0)),
                  pl.BlockSpec((be, na), lambda i: (i, 0)),
                  full(wi), full(wdf), full(wp), full(we), full(wea),
                  full(b1), full(w2), full(b2)],
        out_specs=pl.BlockSpec((be, z), lambda i: (i, 0)),
        out_shape=jax.ShapeDtypeStruct((e, z), _f32),
    )(xs, xd, ea, wi, wdf, wp, we, wea, b1, w2, b2)


def _node_mlp(x, agg_parts, cnt_parts, p, residual):
    n, din = x.shape
    bn = 2000
    W1 = p['Wl1']
    z = W1.shape[1]
    w1x, w1a = W1[:din], W1[din:]
    b1 = p['bl1'].reshape(1, z)
    w2 = p['Wl2']
    b2 = p['bl2'].reshape(1, z)

    def body(x_r, p_r, c_r, w1x_r, w1a_r, b1_r, w2_r, b2_r, o_r):
        xv = x_r[...]
        pv = p_r[...]
        cv = c_r[...]
        cnt = cv[0, :, 0:1] + cv[1, :, 0:1]
        agg = (pv[0] + pv[1]) / jnp.maximum(cnt, 1.0)
        h = jnp.dot(xv, w1x_r[...], preferred_element_type=_f32)
        h = h + jnp.dot(agg, w1a_r[...], preferred_element_type=_f32)
        h = jnp.maximum(h + b1_r[...], 0.0)
        y = jnp.dot(h, w2_r[...], preferred_element_type=_f32) + b2_r[...]
        y = jnp.maximum(y, 0.0)
        if residual:
            y = y + xv
        o_r[...] = y

    full = lambda a: pl.BlockSpec(a.shape, lambda i: tuple(0 for _ in a.shape))
    nz = agg_parts.shape[2]
    return pl.pallas_call(
        body,
        grid=(n // bn,),
        in_specs=[pl.BlockSpec((bn, din), lambda i: (i, 0)),
                  pl.BlockSpec((2, bn, nz), lambda i: (0, i, 0)),
                  pl.BlockSpec((2, bn, 16), lambda i: (0, i, 0)),
                  full(w1x), full(w1a), full(b1), full(w2), full(b2)],
        out_specs=pl.BlockSpec((bn, z), lambda i: (i, 0)),
        out_shape=jax.ShapeDtypeStruct((n, z), _f32),
    )(x, agg_parts, cnt_parts, w1x, w1a, b1, w2, b2)


def _node3_head(x2, x1, agg_parts, cnt_parts, params):
    n, din = x2.shape
    bn = 2000
    p = params['conv3']
    W1 = p['Wl1']
    z = W1.shape[1]
    w1x, w1a = W1[:din], W1[din:]
    b1 = p['bl1'].reshape(1, z)
    w2 = p['Wl2']
    b2 = p['bl2'].reshape(1, z)
    Wf = params['Wf']
    wf1, wf2, wf3 = Wf[0:z], Wf[z:2 * z], Wf[2 * z:3 * z]
    bf = params['bf'].reshape(1, z)
    Wo = params['Wo']
    bo = params['bo'].reshape(1, -1)
    dout = Wo.shape[1]

    def body(x2_r, x1_r, p_r, c_r, w1x_r, w1a_r, b1_r, w2_r, b2_r,
             wf1_r, wf2_r, wf3_r, bf_r, wo_r, bo_r, o_r):
        xv = x2_r[...]
        pv = p_r[...]
        cv = c_r[...]
        cnt = cv[0, :, 0:1] + cv[1, :, 0:1]
        agg = (pv[0] + pv[1]) / jnp.maximum(cnt, 1.0)
        h = jnp.dot(xv, w1x_r[...], preferred_element_type=_f32)
        h = h + jnp.dot(agg, w1a_r[...], preferred_element_type=_f32)
        h = jnp.maximum(h + b1_r[...], 0.0)
        y = jnp.dot(h, w2_r[...], preferred_element_type=_f32) + b2_r[...]
        x3 = jnp.maximum(y, 0.0) + xv
        hz = jnp.dot(x1_r[...], wf1_r[...], preferred_element_type=_f32)
        hz = hz + jnp.dot(xv, wf2_r[...], preferred_element_type=_f32)
        hz = hz + jnp.dot(x3, wf3_r[...], preferred_element_type=_f32)
        hz = jnp.maximum(hz + bf_r[...], 0.0)
        o_r[...] = jnp.dot(hz, wo_r[...],
                           preferred_element_type=_f32) + bo_r[...]

    full = lambda a: pl.BlockSpec(a.shape, lambda i: tuple(0 for _ in a.shape))
    nz = agg_parts.shape[2]
    return pl.pallas_call(
        body,
        grid=(n // bn,),
        in_specs=[pl.BlockSpec((bn, din), lambda i: (i, 0)),
                  pl.BlockSpec((bn, din), lambda i: (i, 0)),
                  pl.BlockSpec((2, bn, nz), lambda i: (0, i, 0)),
                  pl.BlockSpec((2, bn, 16), lambda i: (0, i, 0)),
                  full(w1x), full(w1a), full(b1), full(w2), full(b2),
                  full(wf1), full(wf2), full(wf3), full(bf), full(Wo),
                  full(bo)],
        out_specs=pl.BlockSpec((bn, dout), lambda i: (i, 0)),
        out_shape=jax.ShapeDtypeStruct((n, dout), _f32),
    )(x2, x1, agg_parts, cnt_parts, w1x, w1a, b1, w2, b2,
      wf1, wf2, wf3, bf, Wo, bo)


def kernel(x, edge_index, edge_attr, params):
    src = edge_index[0]
    dst = edge_index[1]
    n, d0 = x.shape

    xs, xd = _gather_pair(x, src, dst)
    m1 = _edge_mlp(xs, xd, edge_attr, params['conv1'], d0)
    agg1, cnt = _segment_parts(m1, dst, n, with_cnt=True)
    x1 = _node_mlp(x, agg1, cnt, params['conv1'], residual=False)

    xs, xd = _gather_pair(x1, src, dst)
    m2 = _edge_mlp(xs, xd, edge_attr, params['conv2'], x1.shape[1])
    agg2 = _segment_parts(m2, dst, n, with_cnt=False)
    x2 = _node_mlp(x1, agg2, cnt, params['conv2'], residual=True)

    xs, xd = _gather_pair(x2, src, dst)
    m3 = _edge_mlp(xs, xd, edge_attr, params['conv3'], x2.shape[1])
    agg3 = _segment_parts(m3, dst, n, with_cnt=False)
    return _node3_head(x2, x1, agg3, cnt, params)


# speedup vs baseline: 4.5983x; 1.5274x over previous
import functools

import jax
import jax.numpy as jnp
from jax import lax
from jax.experimental import pallas as pl
from jax.experimental.pallas import tpu as pltpu
from jax.experimental.pallas import tpu_sc as plsc

_f32 = jnp.float32
_CH = 40
_NCH = 2


def _sc_mesh_info():
    info = plsc.get_sparse_core_info()
    return info.num_cores, info.num_subcores


_K = 5


def _gather_impl(x, eidx, wide, off):
    n, d = x.shape
    if wide:
        e = eidx.size // 2
    else:
        e = eidx.shape[1] // _NCH
    e4 = e // 4
    nc, ns = _sc_mesh_info()
    nw = nc * ns
    ew = e // nw
    nb = ew // (_CH * _K)
    mesh = plsc.VectorSubcoreMesh(core_axis_name="c", subcore_axis_name="s")

    def body(x_hbm, eidx_hbm, xs_hbm, xd_hbm, idx_s, idx_d, *rest):
        rows = rest[:_K]
        sems = rest[_K:]
        c = lax.axis_index("c")
        s = lax.axis_index("s")
        base = (s * nc + c) * ew
        if wide:
            pltpu.sync_copy(eidx_hbm.at[pl.ds(base, ew)], idx_s)
            pltpu.sync_copy(eidx_hbm.at[pl.ds(e + base, ew)], idx_d)
        else:
            pltpu.sync_copy(eidx_hbm.at[0, pl.ds(off + base, ew)], idx_s)
            pltpu.sync_copy(eidx_hbm.at[1, pl.ds(off + base, ew)], idx_d)

        def store(rv, k0, out_hbm):
            if wide:
                j = k0 // e4
                r0 = pl.multiple_of(k0 - j * e4, 8)
                co = pl.multiple_of(j * d, 128)
                pltpu.sync_copy(rv, out_hbm.at[pl.ds(r0, _CH), pl.ds(co, d)])
            else:
                pltpu.sync_copy(rv, out_hbm.at[pl.ds(pl.multiple_of(k0, 8),
                                                     _CH)])

        def step(i, carry):
            w0 = i * (_CH * _K)
            for idx_v, out_hbm in ((idx_s, xs_hbm), (idx_d, xd_hbm)):
                hs = [pltpu.async_copy(
                    x_hbm.at[idx_v.at[pl.ds(w0 + k * _CH, _CH)]],
                    rows[k], sems[k]) for k in range(_K)]
                for k in range(_K):
                    hs[k].wait()
                    store(rows[k], base + w0 + k * _CH, out_hbm)
            return carry

        lax.fori_loop(0, nb, step, 0)

    if wide:
        out_shape = (e4, 4 * d)
        params = None
    else:
        out_shape = (e, d)
        params = pltpu.CompilerParams(use_tc_tiling_on_sc=False)
    f = pl.kernel(
        body,
        mesh=mesh,
        out_type=[jax.ShapeDtypeStruct(out_shape, _f32),
                  jax.ShapeDtypeStruct(out_shape, _f32)],
        scratch_types=([pltpu.VMEM((ew,), jnp.int32),
                        pltpu.VMEM((ew,), jnp.int32)]
                       + [pltpu.VMEM((_CH, d), _f32)] * _K
                       + [pltpu.SemaphoreType.DMA] * _K),
        compiler_params=params,
    )
    return f(x, eidx)


def _gather_wide(x, ppc):
    return _gather_impl(x, ppc.reshape(-1), wide=True, off=0)


def _gather_pair(x, eidx, off):
    return _gather_impl(x, eidx, wide=False, off=off)


def _segment_parts(m, eidx, n, with_cnt, off):
    e, z = m.shape
    nc, ns = _sc_mesh_info()
    nw = nc * ns
    ew = e // nw
    nch = ew // _CH
    rpt = -(-n // (ns * 8)) * 8
    npad = rpt * ns
    mesh = plsc.VectorSubcoreMesh(core_axis_name="c", subcore_axis_name="s")

    out_type = [jax.ShapeDtypeStruct((nc, npad, z), _f32)]
    scratch = [pltpu.VMEM((_CH,), jnp.int32),
               pltpu.VMEM((_CH, z), _f32),
               pltpu.VMEM_SHARED((npad, z), _f32)]
    if with_cnt:
        out_type.append(jax.ShapeDtypeStruct((nc, npad, 16), _f32))
        scratch += [pltpu.VMEM((_CH, 16), _f32),
                    pltpu.VMEM_SHARED((npad, 16), _f32)]

    def body(m_hbm, eidx_hbm, zer_hbm, *rest):
        if with_cnt:
            (z16_hbm, one_hbm, agg_hbm, cnt_hbm,
             idx_v, m_v, acc_sh, one_v, cnt_sh) = rest
        else:
            agg_hbm, idx_v, m_v, acc_sh = rest
        c = lax.axis_index("c")
        s = lax.axis_index("s")
        base = (s * nc + c) * ew
        r0 = s * rpt
        pltpu.sync_copy(zer_hbm.at[pl.ds(r0, rpt)], acc_sh.at[pl.ds(r0, rpt)])
        if with_cnt:
            pltpu.sync_copy(z16_hbm.at[pl.ds(r0, rpt)], cnt_sh.at[pl.ds(r0, rpt)])
            pltpu.sync_copy(one_hbm, one_v)
        plsc.subcore_barrier()

        def step(i, carry):
            mo = pl.multiple_of(base + i * _CH, 8)
            pltpu.sync_copy(eidx_hbm.at[1, pl.ds(off + mo, _CH)], idx_v)
            pltpu.sync_copy(m_hbm.at[pl.ds(mo, _CH)], m_v)
            pltpu.sync_copy(m_v, acc_sh.at[idx_v], add=True)
            if with_cnt:
                pltpu.sync_copy(one_v, cnt_sh.at[idx_v], add=True)
            return carry

        lax.fori_loop(0, nch, step, 0)
        plsc.subcore_barrier()
        pltpu.sync_copy(acc_sh.at[pl.ds(r0, rpt)],
                        agg_hbm.at[c, pl.ds(r0, rpt)])
        if with_cnt:
            pltpu.sync_copy(cnt_sh.at[pl.ds(r0, rpt)],
                            cnt_hbm.at[c, pl.ds(r0, rpt)])

    f = pl.kernel(body, mesh=mesh, out_type=out_type, scratch_types=scratch,
                  compiler_params=pltpu.CompilerParams(
                      use_tc_tiling_on_sc=False))
    zeros = jnp.zeros((npad, z), _f32)
    if with_cnt:
        return f(m, eidx, zeros, jnp.zeros((npad, 16), _f32),
                 jnp.ones((_CH, 16), _f32))
    agg, = f(m, eidx, zeros)
    return agg


def _edge_mlp(xsp, xdp, eacat, p, d):
    e4 = xsp.shape[0]
    be = 800
    W1 = p['We1']
    z = W1.shape[1]
    eye = jnp.eye(4, dtype=_f32)
    kr = lambda w: jnp.kron(eye, w)
    wcat = jnp.concatenate(
        [kr(W1[0:d]), kr(W1[d:2 * d]), kr(W1[2 * d:3 * d])], axis=0)
    we1t = jnp.tile(W1[3 * d:3 * d + 1], (1, 4))
    we2t = jnp.tile(W1[3 * d + 1:3 * d + 2], (1, 4))
    weab = kr(W1[3 * d + 2:])
    b1t = jnp.tile(p['be1'].reshape(1, z), (1, 4))
    w2b = kr(p['We2'])
    b2t = jnp.tile(p['be2'].reshape(1, z), (1, 4))

    def body(xs_r, xd_r, ea_r, wcat_r, we1_r, we2_r, weab_r, b1_r, w2_r,
             b2_r, o_r):
        xsv = xs_r[...]
        xdv = xd_r[...]
        diff = xsv - xdv
        prod = xsv * xdv
        cat = jnp.concatenate([xdv, diff, prod], axis=1)
        acc = jnp.dot(cat, wcat_r[...], preferred_element_type=_f32)
        acc = acc + jnp.dot(ea_r[...], weab_r[...],
                            preferred_element_type=_f32)
        e1s, e2s = [], []
        for j in range(4):
            dj = diff[:, d * j:d * (j + 1)]
            pj = prod[:, d * j:d * (j + 1)]
            e1 = jnp.sqrt(jnp.sum(dj * dj, axis=1, keepdims=True) + 1e-12)
            e2 = jnp.sum(pj, axis=1, keepdims=True)
            e1s.append(jnp.broadcast_to(e1, (be, z)))
            e2s.append(jnp.broadcast_to(e2, (be, z)))
        acc = acc + jnp.concatenate(e1s, axis=1) * we1_r[...]
        acc = acc + jnp.concatenate(e2s, axis=1) * we2_r[...]
        h = jnp.maximum(acc + b1_r[...], 0.0)
        o = jnp.dot(h, w2_r[...], preferred_element_type=_f32) + b2_r[...]
        o_r[...] = jnp.maximum(o, 0.0)

    full = lambda a: pl.BlockSpec(a.shape, lambda i: (0, 0))
    return pl.pallas_call(
        body,
        grid=(e4 // be,),
        in_specs=[pl.BlockSpec((be, 4 * d), lambda i: (i, 0)),
                  pl.BlockSpec((be, 4 * d), lambda i: (i, 0)),
                  pl.BlockSpec((be, 16), lambda i: (i, 0)),
                  full(wcat), full(we1t), full(we2t), full(weab),
                  full(b1t), full(w2b), full(b2t)],
        out_specs=pl.BlockSpec((be, 4 * z), lambda i: (i, 0)),
        out_shape=jax.ShapeDtypeStruct((e4, 4 * z), _f32),
    )(xsp, xdp, eacat, wcat, we1t, we2t, weab, b1t, w2b, b2t)


def _node_mlp(x, agg_parts, cnt_parts, p, residual):
    n, din = x.shape
    bn = 2000
    W1 = p['Wl1']
    z = W1.shape[1]
    w1x, w1a = W1[:din], W1[din:]
    b1 = p['bl1'].reshape(1, z)
    w2 = p['Wl2']
    b2 = p['bl2'].reshape(1, z)
    np_ = len(agg_parts)
    nc_ = len(cnt_parts)

    def body(*refs):
        x_r = refs[0]
        p_rs = refs[1:1 + np_]
        c_rs = refs[1 + np_:1 + np_ + nc_]
        w1x_r, w1a_r, b1_r, w2_r, b2_r, o_r = refs[1 + np_ + nc_:]
        xv = x_r[...]
        cnt = sum(jnp.sum(c_r[...][:, :, 0:1], axis=0) for c_r in c_rs)
        agg = sum(jnp.sum(p_r[...], axis=0) for p_r in p_rs)
        agg = agg / jnp.maximum(cnt, 1.0)
        h = jnp.dot(xv, w1x_r[...], preferred_element_type=_f32)
        h = h + jnp.dot(agg, w1a_r[...], preferred_element_type=_f32)
        h = jnp.maximum(h + b1_r[...], 0.0)
        y = jnp.dot(h, w2_r[...], preferred_element_type=_f32) + b2_r[...]
        y = jnp.maximum(y, 0.0)
        if residual:
            y = y + xv
        o_r[...] = y

    full = lambda a: pl.BlockSpec(a.shape, lambda i: tuple(0 for _ in a.shape))
    nz = agg_parts[0].shape[2]
    pspec = pl.BlockSpec((2, bn, nz), lambda i: (0, i, 0))
    cspec = pl.BlockSpec((2, bn, 16), lambda i: (0, i, 0))
    return pl.pallas_call(
        body,
        grid=(n // bn,),
        in_specs=([pl.BlockSpec((bn, din), lambda i: (i, 0))]
                  + [pspec] * np_ + [cspec] * nc_
                  + [full(w1x), full(w1a), full(b1), full(w2), full(b2)]),
        out_specs=pl.BlockSpec((bn, z), lambda i: (i, 0)),
        out_shape=jax.ShapeDtypeStruct((n, z), _f32),
    )(x, *agg_parts, *cnt_parts, w1x, w1a, b1, w2, b2)


def _node3_head(x2, x1, agg_parts, cnt_parts, params):
    n, din = x2.shape
    bn = 2000
    p = params['conv3']
    W1 = p['Wl1']
    z = W1.shape[1]
    w1x, w1a = W1[:din], W1[din:]
    b1 = p['bl1'].reshape(1, z)
    w2 = p['Wl2']
    b2 = p['bl2'].reshape(1, z)
    Wf = params['Wf']
    wf1, wf2, wf3 = Wf[0:z], Wf[z:2 * z], Wf[2 * z:3 * z]
    bf = params['bf'].reshape(1, z)
    Wo = params['Wo']
    bo = params['bo'].reshape(1, -1)
    dout = Wo.shape[1]
    np_ = len(agg_parts)
    nc_ = len(cnt_parts)

    def body(*refs):
        x2_r, x1_r = refs[0], refs[1]
        p_rs = refs[2:2 + np_]
        c_rs = refs[2 + np_:2 + np_ + nc_]
        (w1x_r, w1a_r, b1_r, w2_r, b2_r, wf1_r, wf2_r, wf3_r, bf_r,
         wo_r, bo_r, o_r) = refs[2 + np_ + nc_:]
        xv = x2_r[...]
        cnt = sum(jnp.sum(c_r[...][:, :, 0:1], axis=0) for c_r in c_rs)
        agg = sum(jnp.sum(p_r[...], axis=0) for p_r in p_rs)
        agg = agg / jnp.maximum(cnt, 1.0)
        h = jnp.dot(xv, w1x_r[...], preferred_element_type=_f32)
        h = h + jnp.dot(agg, w1a_r[...], preferred_element_type=_f32)
        h = jnp.maximum(h + b1_r[...], 0.0)
        y = jnp.dot(h, w2_r[...], preferred_element_type=_f32) + b2_r[...]
        x3 = jnp.maximum(y, 0.0) + xv
        hz = jnp.dot(x1_r[...], wf1_r[...], preferred_element_type=_f32)
        hz = hz + jnp.dot(xv, wf2_r[...], preferred_element_type=_f32)
        hz = hz + jnp.dot(x3, wf3_r[...], preferred_element_type=_f32)
        hz = jnp.maximum(hz + bf_r[...], 0.0)
        o_r[...] = jnp.dot(hz, wo_r[...],
                           preferred_element_type=_f32) + bo_r[...]

    full = lambda a: pl.BlockSpec(a.shape, lambda i: tuple(0 for _ in a.shape))
    nz = agg_parts[0].shape[2]
    pspec = pl.BlockSpec((2, bn, nz), lambda i: (0, i, 0))
    cspec = pl.BlockSpec((2, bn, 16), lambda i: (0, i, 0))
    return pl.pallas_call(
        body,
        grid=(n // bn,),
        in_specs=([pl.BlockSpec((bn, din), lambda i: (i, 0)),
                   pl.BlockSpec((bn, din), lambda i: (i, 0))]
                  + [pspec] * np_ + [cspec] * nc_
                  + [full(w1x), full(w1a), full(b1), full(w2), full(b2),
                     full(wf1), full(wf2), full(wf3), full(bf), full(Wo),
                     full(bo)]),
        out_specs=pl.BlockSpec((bn, dout), lambda i: (i, 0)),
        out_shape=jax.ShapeDtypeStruct((n, dout), _f32),
    )(x2, x1, *agg_parts, *cnt_parts, w1x, w1a, b1, w2, b2,
      wf1, wf2, wf3, bf, Wo, bo)


def kernel(x, edge_index, edge_attr, params):
    n, d0 = x.shape
    e = edge_index.shape[1]
    e4 = e // 4
    eh = e // _NCH
    e4h = e4 // _NCH
    eacat = edge_attr.reshape(e4, 16)
    ppch = (edge_index.reshape(2, _NCH, e4h, 4)
            .transpose(1, 0, 3, 2).reshape(_NCH, 2 * eh))

    d1 = params['conv1']['Wl1'].shape[1]

    ms = []
    for c in range(_NCH):
        xsp, xdp = _gather_wide(x, ppch[c])
        ms.append(_edge_mlp(xsp, xdp, eacat[c * e4h:(c + 1) * e4h],
                            params['conv1'], d0))
    aggs, cnts = [], []
    for c in range(_NCH):
        a, ct = _segment_parts(ms[c].reshape(eh, 32), edge_index, n,
                               with_cnt=True, off=c * eh)
        aggs.append(a)
        cnts.append(ct)
    x1 = _node_mlp(x, aggs, cnts, params['conv1'], residual=False)

    ms = []
    for c in range(_NCH):
        xs, xd = _gather_pair(x1, edge_index, off=c * eh)
        ms.append(_edge_mlp(xs.reshape(e4h, 4 * d1), xd.reshape(e4h, 4 * d1),
                            eacat[c * e4h:(c + 1) * e4h],
                            params['conv2'], d1))
    aggs = [_segment_parts(ms[c].reshape(eh, 32), edge_index, n,
                           with_cnt=False, off=c * eh) for c in range(_NCH)]
    x2 = _node_mlp(x1, aggs, cnts, params['conv2'], residual=True)

    ms = []
    for c in range(_NCH):
        xs, xd = _gather_pair(x2, edge_index, off=c * eh)
        ms.append(_edge_mlp(xs.reshape(e4h, 4 * d1), xd.reshape(e4h, 4 * d1),
                            eacat[c * e4h:(c + 1) * e4h],
                            params['conv3'], d1))
    aggs = [_segment_parts(ms[c].reshape(eh, 32), edge_index, n,
                           with_cnt=False, off=c * eh) for c in range(_NCH)]
    return _node3_head(x2, x1, aggs, cnts, params)
